# baseline (speedup 1.0000x reference)
import jax
import jax.numpy as jnp
from jax.experimental import pallas as pl
from jax.experimental.pallas import tpu as pltpu

_N = 22500
_T = 22
_NPAD = _T * 8 * 128
_MAX_OUT = 300
_IOU_THR = 0.7
_IMG = 800.0


def _pick(a, b):
    take_b = (b[0] > a[0]) | ((b[0] == a[0]) & (b[1] < a[1]))
    return tuple(jnp.where(take_b, y, x) for x, y in zip(a, b))


def _nms_body(scores_ref, reg_ref, anc_ref, out_ref, box_ref):
    f0 = jnp.float32(0.0)
    x1a = anc_ref[0]
    y1a = anc_ref[1]
    x2a = anc_ref[2]
    y2a = anc_ref[3]
    wa = x2a - x1a
    ha = y2a - y1a
    cxa = x1a + wa * 0.5
    cya = y1a + ha * 0.5
    cx = reg_ref[0] * wa + cxa
    cy = reg_ref[1] * ha + cya
    w = wa * jnp.exp(reg_ref[2])
    h = ha * jnp.exp(reg_ref[3])
    x1 = jnp.minimum(jnp.maximum(cx - w * 0.5, f0), _IMG)
    y1 = jnp.minimum(jnp.maximum(cy - h * 0.5, f0), _IMG)
    x2 = jnp.minimum(jnp.maximum(cx + w * 0.5, f0), _IMG)
    y2 = jnp.minimum(jnp.maximum(cy + h * 0.5, f0), _IMG)
    box_ref[0] = x1
    box_ref[1] = y1
    box_ref[2] = x2
    box_ref[3] = y2
    box_ref[4] = (x2 - x1) * (y2 - y1)

    it = jax.lax.broadcasted_iota
    rowi = (it(jnp.int32, (_T, 8, 128), 0) * 8
            + it(jnp.int32, (_T, 8, 128), 1))
    lane = it(jnp.int32, (1, 128), 1)

    def step(i, s):
        x1c = box_ref[0]
        y1c = box_ref[1]
        x2c = box_ref[2]
        y2c = box_ref[3]
        so = scores_ref[...]
        items = [(s[t], rowi[t], x1c[t], y1c[t], x2c[t], y2c[t], so[t])
                 for t in range(_T)]
        while len(items) > 1:
            nxt = [_pick(items[j], items[j + 1])
                   for j in range(0, len(items) - 1, 2)]
            if len(items) % 2:
                nxt.append(items[-1])
            items = nxt
        cur = items[0]
        for sh in (1, 2, 4):
            cur = _pick(cur, tuple(pltpu.roll(v, sh, 0) for v in cur))
        vrow = cur[0][0:1]
        m = jnp.max(vrow, keepdims=True)
        lk = jnp.where(vrow == m, lane, _NPAD)
        lw = jnp.min(lk, keepdims=True)
        onehot = lane == lw
        x1s = jnp.sum(jnp.where(onehot, cur[2][0:1], f0), keepdims=True)
        y1s = jnp.sum(jnp.where(onehot, cur[3][0:1], f0), keepdims=True)
        x2s = jnp.sum(jnp.where(onehot, cur[4][0:1], f0), keepdims=True)
        y2s = jnp.sum(jnp.where(onehot, cur[5][0:1], f0), keepdims=True)
        ss = jnp.sum(jnp.where(onehot, cur[6][0:1], f0), keepdims=True)
        area_s = (x2s - x1s) * (y2s - y1s)
        xx1 = jnp.maximum(x1c, x1s[None])
        yy1 = jnp.maximum(y1c, y1s[None])
        xx2 = jnp.minimum(x2c, x2s[None])
        yy2 = jnp.minimum(y2c, y2s[None])
        inter = jnp.maximum(xx2 - xx1, f0) * jnp.maximum(yy2 - yy1, f0)
        iou = inter / (box_ref[4] + area_s[None] - inter + 1e-9)
        s2 = jnp.where(iou >= _IOU_THR, -1e9, s)
        row = jnp.where(lane == 0, x1s,
              jnp.where(lane == 1, y1s,
              jnp.where(lane == 2, x2s,
              jnp.where(lane == 3, y2s, ss))))
        out_ref[pl.ds(i, 1), :] = row
        return s2

    jax.lax.fori_loop(0, _MAX_OUT, step, scores_ref[...])


def _to_colmajor(a):
    return a.reshape(128, _T, 8).transpose(1, 2, 0)


def kernel(cls_output, reg_output, anchors):
    f32 = jnp.float32
    pad = _NPAD - _N
    scores = _to_colmajor(jnp.concatenate(
        [cls_output.astype(f32), jnp.full((pad,), -jnp.inf, f32)]))
    reg_p = jnp.concatenate(
        [reg_output.astype(f32), jnp.zeros((pad, 4), f32)]).T
    anc_p = jnp.concatenate(
        [anchors.astype(f32), jnp.zeros((pad, 4), f32)]).T
    reg4 = jnp.stack([_to_colmajor(reg_p[k]) for k in range(4)])
    anc4 = jnp.stack([_to_colmajor(anc_p[k]) for k in range(4)])

    out = pl.pallas_call(
        _nms_body,
        out_shape=jax.ShapeDtypeStruct((304, 128), f32),
        scratch_shapes=[
            pltpu.VMEM((5, _T, 8, 128), f32),
        ],
    )(scores, reg4, anc4)

    rois = out[:_MAX_OUT, 0:4]
    roi_scores = out[:_MAX_OUT, 4]
    return roi_scores, rois


# speedup vs baseline: 1.4895x; 1.4895x over previous
import jax
import jax.numpy as jnp
from jax.experimental import pallas as pl
from jax.experimental.pallas import tpu as pltpu

_N = 22500
_T = 22
_NPAD = _T * 8 * 128
_MAX_OUT = 300
_IOU_THR = 0.7
_IMG = 800.0


def _pick(a, b):
    take_b = (b[0] > a[0]) | ((b[0] == a[0]) & (b[1] < a[1]))
    return tuple(jnp.where(take_b, y, x) for x, y in zip(a, b))


def _nms_body(scores_ref, reg_ref, anc_ref, out_ref, box_ref, sel_ref):
    f0 = jnp.float32(0.0)
    x1a = anc_ref[0]
    y1a = anc_ref[1]
    x2a = anc_ref[2]
    y2a = anc_ref[3]
    wa = x2a - x1a
    ha = y2a - y1a
    cxa = x1a + wa * 0.5
    cya = y1a + ha * 0.5
    cx = reg_ref[0] * wa + cxa
    cy = reg_ref[1] * ha + cya
    w = wa * jnp.exp(reg_ref[2])
    h = ha * jnp.exp(reg_ref[3])
    x1 = jnp.minimum(jnp.maximum(cx - w * 0.5, f0), _IMG)
    y1 = jnp.minimum(jnp.maximum(cy - h * 0.5, f0), _IMG)
    x2 = jnp.minimum(jnp.maximum(cx + w * 0.5, f0), _IMG)
    y2 = jnp.minimum(jnp.maximum(cy + h * 0.5, f0), _IMG)
    box_ref[0] = x1
    box_ref[1] = y1
    box_ref[2] = x2
    box_ref[3] = y2
    box_ref[4] = (x2 - x1) * (y2 - y1)

    it = jax.lax.broadcasted_iota
    rowi = (it(jnp.int32, (_T, 8, 128), 0) * 8
            + it(jnp.int32, (_T, 8, 128), 1))
    lane = it(jnp.int32, (1, 128), 1)
    lanef = lane.astype(jnp.float32)

    def step(i, s):
        x1c = box_ref[0]
        y1c = box_ref[1]
        x2c = box_ref[2]
        y2c = box_ref[3]
        so = scores_ref[...]
        items = [(s[t], rowi[t], x1c[t], y1c[t], x2c[t], y2c[t], so[t])
                 for t in range(_T)]
        while len(items) > 1:
            nxt = [_pick(items[j], items[j + 1])
                   for j in range(0, len(items) - 1, 2)]
            if len(items) % 2:
                nxt.append(items[-1])
            items = nxt
        cur = items[0]
        for sh in (1, 2, 4):
            cur = _pick(cur, tuple(pltpu.roll(v, sh, 0) for v in cur))
        vrow = cur[0][0:1]
        m = jnp.max(vrow, keepdims=True)
        mask = vrow == m
        cnt = jnp.sum(jnp.where(mask, 1.0, f0), keepdims=True)
        x1s = jnp.sum(jnp.where(mask, cur[2][0:1], f0), keepdims=True)
        y1s = jnp.sum(jnp.where(mask, cur[3][0:1], f0), keepdims=True)
        x2s = jnp.sum(jnp.where(mask, cur[4][0:1], f0), keepdims=True)
        y2s = jnp.sum(jnp.where(mask, cur[5][0:1], f0), keepdims=True)
        ss = jnp.sum(jnp.where(mask, cur[6][0:1], f0), keepdims=True)
        sel_ref[0:1, :] = jnp.broadcast_to(x1s, (1, 128))
        sel_ref[1:2, :] = jnp.broadcast_to(y1s, (1, 128))
        sel_ref[2:3, :] = jnp.broadcast_to(x2s, (1, 128))
        sel_ref[3:4, :] = jnp.broadcast_to(y2s, (1, 128))
        sel_ref[4:5, :] = jnp.broadcast_to(ss, (1, 128))

        @pl.when(cnt[0, 0] > 1.0)
        def _tie_fix():
            lw = jnp.min(jnp.where(mask, lanef, 1e9), keepdims=True)
            onehot = lanef == lw
            sel_ref[0:1, :] = jnp.broadcast_to(jnp.sum(
                jnp.where(onehot, cur[2][0:1], f0), keepdims=True), (1, 128))
            sel_ref[1:2, :] = jnp.broadcast_to(jnp.sum(
                jnp.where(onehot, cur[3][0:1], f0), keepdims=True), (1, 128))
            sel_ref[2:3, :] = jnp.broadcast_to(jnp.sum(
                jnp.where(onehot, cur[4][0:1], f0), keepdims=True), (1, 128))
            sel_ref[3:4, :] = jnp.broadcast_to(jnp.sum(
                jnp.where(onehot, cur[5][0:1], f0), keepdims=True), (1, 128))
            sel_ref[4:5, :] = jnp.broadcast_to(jnp.sum(
                jnp.where(onehot, cur[6][0:1], f0), keepdims=True), (1, 128))

        x1b = sel_ref[0:1, :]
        y1b = sel_ref[1:2, :]
        x2b = sel_ref[2:3, :]
        y2b = sel_ref[3:4, :]
        ssb = sel_ref[4:5, :]
        area_s = (x2b - x1b) * (y2b - y1b)
        xx1 = jnp.maximum(x1c, x1b[None])
        yy1 = jnp.maximum(y1c, y1b[None])
        xx2 = jnp.minimum(x2c, x2b[None])
        yy2 = jnp.minimum(y2c, y2b[None])
        inter = jnp.maximum(xx2 - xx1, f0) * jnp.maximum(yy2 - yy1, f0)
        iou = inter / (box_ref[4] + area_s[None] - inter + 1e-9)
        s2 = jnp.where(iou >= _IOU_THR, -1e9, s)
        row = jnp.where(lane == 0, x1b,
              jnp.where(lane == 1, y1b,
              jnp.where(lane == 2, x2b,
              jnp.where(lane == 3, y2b, ssb))))
        out_ref[pl.ds(i, 1), :] = row
        return s2

    jax.lax.fori_loop(0, _MAX_OUT, step, scores_ref[...])


def _to_colmajor(a):
    return a.reshape(128, _T, 8).transpose(1, 2, 0)


def kernel(cls_output, reg_output, anchors):
    f32 = jnp.float32
    pad = _NPAD - _N
    scores = _to_colmajor(jnp.concatenate(
        [cls_output.astype(f32), jnp.full((pad,), -jnp.inf, f32)]))
    reg_p = jnp.concatenate(
        [reg_output.astype(f32), jnp.zeros((pad, 4), f32)]).T
    anc_p = jnp.concatenate(
        [anchors.astype(f32), jnp.zeros((pad, 4), f32)]).T
    reg4 = jnp.stack([_to_colmajor(reg_p[k]) for k in range(4)])
    anc4 = jnp.stack([_to_colmajor(anc_p[k]) for k in range(4)])

    out = pl.pallas_call(
        _nms_body,
        out_shape=jax.ShapeDtypeStruct((304, 128), f32),
        scratch_shapes=[
            pltpu.VMEM((5, _T, 8, 128), f32),
            pltpu.VMEM((8, 128), f32),
        ],
    )(scores, reg4, anc4)

    rois = out[:_MAX_OUT, 0:4]
    roi_scores = out[:_MAX_OUT, 4]
    return roi_scores, rois
